# baseline (speedup 1.0000x reference)
import jax
import jax.numpy as jnp
from jax.experimental import pallas as pl
from jax.experimental.pallas import tpu as pltpu

_N = 20000
_ROWS = 160
_COLS = 128
_N_PAD = _ROWS * _COLS
_SCORE_THRESH = 0.05
_NMS_THRESH = 0.5
_K = 100
_NEG = -1e9


def _nms_body(x1_ref, y1_ref, x2_ref, y2_ref, s_ref, out_ref):
    x1 = x1_ref[...]
    y1 = y1_ref[...]
    x2 = x2_ref[...]
    y2 = y2_ref[...]
    scores = s_ref[...]
    s0 = jnp.where(scores > _SCORE_THRESH, scores, _NEG)
    area3 = (x2 - x1) * (y2 - y1) * (1.0 / 3.0)
    rowid = jax.lax.broadcasted_iota(jnp.int32, (_ROWS, _COLS), 0)
    lane = jax.lax.broadcasted_iota(jnp.int32, (1, _COLS), 1)
    ones_mat = jnp.ones((_COLS, _COLS), jnp.float32)
    nchunk = _ROWS // 8

    def body(i, s):
        ps = [
            (s[k * 8:(k + 1) * 8], rowid[k * 8:(k + 1) * 8])
            for k in range(nchunk)
        ]
        while len(ps) > 1:
            nxt = []
            for a in range(0, len(ps) - 1, 2):
                pa, pb = ps[a], ps[a + 1]
                take = (pa[0] > pb[0]) | ((pa[0] == pb[0]) & (pa[1] < pb[1]))
                nxt.append(tuple(jnp.where(take, u, v) for u, v in zip(pa, pb)))
            if len(ps) % 2:
                nxt.append(ps[-1])
            ps = nxt
        w8 = ps[0]
        t4 = [u[0:4] for u in w8]
        b4 = [u[4:8] for u in w8]
        take = (t4[0] > b4[0]) | ((t4[0] == b4[0]) & (t4[1] < b4[1]))
        w4 = [jnp.where(take, u, v) for u, v in zip(t4, b4)]
        t2 = [u[0:2] for u in w4]
        b2 = [u[2:4] for u in w4]
        take = (t2[0] > b2[0]) | ((t2[0] == b2[0]) & (t2[1] < b2[1]))
        w2 = [jnp.where(take, u, v) for u, v in zip(t2, b2)]
        t1 = [u[0:1] for u in w2]
        b1 = [u[1:2] for u in w2]
        take = (t1[0] > b1[0]) | ((t1[0] == b1[0]) & (t1[1] < b1[1]))
        w1 = [jnp.where(take, u, v) for u, v in zip(t1, b1)]
        colv, colrow = w1

        hot2d = rowid == colrow
        colx1 = jnp.sum(jnp.where(hot2d, x1, 0.0), axis=0, keepdims=True)
        coly1 = jnp.sum(jnp.where(hot2d, y1, 0.0), axis=0, keepdims=True)
        colx2 = jnp.sum(jnp.where(hot2d, x2, 0.0), axis=0, keepdims=True)
        coly2 = jnp.sum(jnp.where(hot2d, y2, 0.0), axis=0, keepdims=True)

        c = jnp.argmax(colv, axis=1).reshape(1, 1)
        lane_hot = lane == c

        stack = jnp.concatenate(
            [colx1, coly1, colx2, coly2, colv], axis=0
        )
        t = jnp.where(lane_hot, stack, 0.0)
        b = jax.lax.dot_general(
            t, ones_mat, (((1,), (0,)), ((), ())),
            preferred_element_type=jnp.float32,
            precision=jax.lax.Precision.HIGHEST,
        )
        bx1 = b[0:1, :]
        by1 = b[1:2, :]
        bx2 = b[2:3, :]
        by2 = b[3:4, :]
        m = b[4:5, :]
        barea3 = (bx2 - bx1) * (by2 - by1) * (1.0 / 3.0)
        valid = m > _NEG / 2.0

        iw = jnp.maximum(jnp.minimum(bx2, x2) - jnp.maximum(bx1, x1), 0.0)
        ih = jnp.maximum(jnp.minimum(by2, y2) - jnp.maximum(by1, y1), 0.0)
        inter = iw * ih
        suppress = inter > area3 + (barea3 + 1e-9 / 3.0)
        s = jnp.where(suppress, _NEG, s)

        row = (
            jnp.where(lane == 0, bx1, 0.0)
            + jnp.where(lane == 1, by1, 0.0)
            + jnp.where(lane == 2, bx2, 0.0)
            + jnp.where(lane == 3, by2, 0.0)
            + jnp.where(lane == 4, m, 0.0)
        )
        out_ref[pl.ds(i, 1), :] = jnp.where(valid, row, 0.0)
        return s

    jax.lax.fori_loop(0, _K, body, s0, unroll=2)


def _to_layout(v, fill):
    v = jnp.pad(v, (0, _N_PAD - _N), constant_values=fill)
    return v.reshape(_COLS, _ROWS).T[:, ::-1]


def kernel(boxes, scores):
    x1 = _to_layout(boxes[:, 0], 0.0)
    y1 = _to_layout(boxes[:, 1], 0.0)
    x2 = _to_layout(boxes[:, 2], 0.0)
    y2 = _to_layout(boxes[:, 3], 0.0)
    s = _to_layout(scores, -1.0)

    out = pl.pallas_call(
        _nms_body,
        out_shape=jax.ShapeDtypeStruct((_K, _COLS), jnp.float32),
        in_specs=[pl.BlockSpec(memory_space=pltpu.VMEM)] * 5,
        out_specs=pl.BlockSpec(memory_space=pltpu.VMEM),
    )(x1, y1, x2, y2, s)
    return out[:, :5]


# speedup vs baseline: 1.2076x; 1.2076x over previous
import jax
import jax.numpy as jnp
from jax.experimental import pallas as pl
from jax.experimental.pallas import tpu as pltpu

_N = 20000
_ROWS = 160
_COLS = 128
_N_PAD = _ROWS * _COLS
_POOL = 8
_SCORE_THRESH = 0.05
_NMS_THRESH = 0.5
_K = 100
_NEG = -1e9


def _take(pa, pb):
    t = (pa[0] > pb[0]) | ((pa[0] == pb[0]) & (pa[1] < pb[1]))
    return tuple(jnp.where(t, u, v) for u, v in zip(pa, pb))


def _nms_body(x1_ref, y1_ref, x2_ref, y2_ref, s_ref, out_ref):
    x1 = x1_ref[...]
    y1 = y1_ref[...]
    x2 = x2_ref[...]
    y2 = y2_ref[...]
    scores = s_ref[...]
    s0 = jnp.where(scores > _SCORE_THRESH, scores, _NEG)
    area3 = (x2 - x1) * (y2 - y1) * (1.0 / 3.0)
    rowid = jax.lax.broadcasted_iota(jnp.int32, (_ROWS, _COLS), 0)
    lane = jax.lax.broadcasted_iota(jnp.int32, (1, _COLS), 1)
    ones_mat = jnp.ones((_COLS, _COLS), jnp.float32)
    nchunk = _ROWS // 8

    sm = s0
    slots = []
    for _ in range(_POOL):
        cmax = jnp.max(sm, axis=0, keepdims=True)
        rsel = jnp.min(
            jnp.where(sm == cmax, rowid, _ROWS), axis=0, keepdims=True
        )
        hot = rowid == rsel
        px1 = jnp.sum(jnp.where(hot, x1, 0.0), axis=0, keepdims=True)
        py1 = jnp.sum(jnp.where(hot, y1, 0.0), axis=0, keepdims=True)
        px2 = jnp.sum(jnp.where(hot, x2, 0.0), axis=0, keepdims=True)
        py2 = jnp.sum(jnp.where(hot, y2, 0.0), axis=0, keepdims=True)
        slots.append((cmax, rsel.astype(jnp.float32), px1, py1, px2, py2))
        sm = jnp.where(hot, _NEG, sm)
    t_next = jnp.max(sm, axis=0, keepdims=True)
    t_next = jnp.max(t_next, axis=1, keepdims=True)

    pool_s = jnp.concatenate([sl[0] for sl in slots], axis=0)
    pool_r = jnp.concatenate([sl[1] for sl in slots], axis=0)
    pool_x1 = jnp.concatenate([sl[2] for sl in slots], axis=0)
    pool_y1 = jnp.concatenate([sl[3] for sl in slots], axis=0)
    pool_x2 = jnp.concatenate([sl[4] for sl in slots], axis=0)
    pool_y2 = jnp.concatenate([sl[5] for sl in slots], axis=0)
    pool_a3 = (pool_x2 - pool_x1) * (pool_y2 - pool_y1) * (1.0 / 3.0)

    def fast_body(i, st):
        ps, pois = st
        pair = (ps, pool_r, pool_x1, pool_y1, pool_x2, pool_y2)
        w4 = _take(tuple(u[0:4] for u in pair), tuple(u[4:8] for u in pair))
        w2 = _take(tuple(u[0:2] for u in w4), tuple(u[2:4] for u in w4))
        w1 = _take(tuple(u[0:1] for u in w2), tuple(u[1:2] for u in w2))
        colv, colr, colx1, coly1, colx2, coly2 = w1

        c = jnp.argmax(colv, axis=1).reshape(1, 1)
        lane_hot = lane == c
        stack = jnp.concatenate(
            [colx1, coly1, colx2, coly2, colv, colr], axis=0
        )
        t = jnp.where(lane_hot, stack, 0.0)
        b = jax.lax.dot_general(
            t, ones_mat, (((1,), (0,)), ((), ())),
            preferred_element_type=jnp.float32,
            precision=jax.lax.Precision.HIGHEST,
        )
        bx1 = b[0:1, :]
        by1 = b[1:2, :]
        bx2 = b[2:3, :]
        by2 = b[3:4, :]
        m = b[4:5, :]
        wrow = b[5:6, :]
        barea3 = (bx2 - bx1) * (by2 - by1) * (1.0 / 3.0)
        valid = m > _NEG / 2.0

        iw = jnp.maximum(
            jnp.minimum(bx2, pool_x2) - jnp.maximum(bx1, pool_x1), 0.0
        )
        ih = jnp.maximum(
            jnp.minimum(by2, pool_y2) - jnp.maximum(by1, pool_y1), 0.0
        )
        inter = iw * ih
        suppress = inter > pool_a3 + (barea3 + 1e-9 / 3.0)
        retire = lane_hot & (ps == m) & (pool_r == wrow)
        ps = jnp.where(suppress | retire, _NEG, ps)

        bad = (m[0:1, 0:1] <= t_next) & (t_next > _NEG / 2.0)
        pois = jnp.maximum(pois, bad.astype(jnp.float32))

        row = (
            jnp.where(lane == 0, bx1, 0.0)
            + jnp.where(lane == 1, by1, 0.0)
            + jnp.where(lane == 2, bx2, 0.0)
            + jnp.where(lane == 3, by2, 0.0)
            + jnp.where(lane == 4, m, 0.0)
        )
        out_ref[pl.ds(i, 1), :] = jnp.where(valid, row, 0.0)
        return ps, pois

    pois0 = jnp.zeros((1, 1), jnp.float32)
    _, pois = jax.lax.fori_loop(
        0, _K, fast_body, (pool_s, pois0), unroll=False
    )

    @pl.when(pois.astype(jnp.int32)[0, 0] != 0)
    def _slow():
        def body(i, s):
            ps = [
                (s[k * 8:(k + 1) * 8], rowid[k * 8:(k + 1) * 8])
                for k in range(nchunk)
            ]
            while len(ps) > 1:
                nxt = []
                for a in range(0, len(ps) - 1, 2):
                    nxt.append(_take(ps[a], ps[a + 1]))
                if len(ps) % 2:
                    nxt.append(ps[-1])
                ps = nxt
            w8 = ps[0]
            w4 = _take(tuple(u[0:4] for u in w8), tuple(u[4:8] for u in w8))
            w2 = _take(tuple(u[0:2] for u in w4), tuple(u[2:4] for u in w4))
            w1 = _take(tuple(u[0:1] for u in w2), tuple(u[1:2] for u in w2))
            colv, colrow = w1

            hot2d = rowid == colrow
            colx1 = jnp.sum(jnp.where(hot2d, x1, 0.0), axis=0, keepdims=True)
            coly1 = jnp.sum(jnp.where(hot2d, y1, 0.0), axis=0, keepdims=True)
            colx2 = jnp.sum(jnp.where(hot2d, x2, 0.0), axis=0, keepdims=True)
            coly2 = jnp.sum(jnp.where(hot2d, y2, 0.0), axis=0, keepdims=True)

            c = jnp.argmax(colv, axis=1).reshape(1, 1)
            lane_hot = lane == c
            stack = jnp.concatenate(
                [colx1, coly1, colx2, coly2, colv], axis=0
            )
            t = jnp.where(lane_hot, stack, 0.0)
            b = jax.lax.dot_general(
                t, ones_mat, (((1,), (0,)), ((), ())),
                preferred_element_type=jnp.float32,
                precision=jax.lax.Precision.HIGHEST,
            )
            bx1 = b[0:1, :]
            by1 = b[1:2, :]
            bx2 = b[2:3, :]
            by2 = b[3:4, :]
            m = b[4:5, :]
            barea3 = (bx2 - bx1) * (by2 - by1) * (1.0 / 3.0)
            valid = m > _NEG / 2.0

            iw = jnp.maximum(jnp.minimum(bx2, x2) - jnp.maximum(bx1, x1), 0.0)
            ih = jnp.maximum(jnp.minimum(by2, y2) - jnp.maximum(by1, y1), 0.0)
            inter = iw * ih
            suppress = inter > area3 + (barea3 + 1e-9 / 3.0)
            s = jnp.where(suppress, _NEG, s)

            row = (
                jnp.where(lane == 0, bx1, 0.0)
                + jnp.where(lane == 1, by1, 0.0)
                + jnp.where(lane == 2, bx2, 0.0)
                + jnp.where(lane == 3, by2, 0.0)
                + jnp.where(lane == 4, m, 0.0)
            )
            out_ref[pl.ds(i, 1), :] = jnp.where(valid, row, 0.0)
            return s

        jax.lax.fori_loop(0, _K, body, s0, unroll=False)


def _to_layout(v, fill):
    v = jnp.pad(v, (0, _N_PAD - _N), constant_values=fill)
    return v.reshape(_COLS, _ROWS).T[:, ::-1]


def kernel(boxes, scores):
    x1 = _to_layout(boxes[:, 0], 0.0)
    y1 = _to_layout(boxes[:, 1], 0.0)
    x2 = _to_layout(boxes[:, 2], 0.0)
    y2 = _to_layout(boxes[:, 3], 0.0)
    s = _to_layout(scores, -1.0)

    out = pl.pallas_call(
        _nms_body,
        out_shape=jax.ShapeDtypeStruct((_K, _COLS), jnp.float32),
        in_specs=[pl.BlockSpec(memory_space=pltpu.VMEM)] * 5,
        out_specs=pl.BlockSpec(memory_space=pltpu.VMEM),
    )(x1, y1, x2, y2, s)
    return out[:, :5]


# speedup vs baseline: 1.2737x; 1.0548x over previous
import jax
import jax.numpy as jnp
from jax.experimental import pallas as pl
from jax.experimental.pallas import tpu as pltpu

_N = 20000
_ROWS = 160
_COLS = 128
_N_PAD = _ROWS * _COLS
_POOL = 8
_SCORE_THRESH = 0.05
_NMS_THRESH = 0.5
_K = 100
_NEG = -1e9


def _take(pa, pb):
    t = (pa[0] > pb[0]) | ((pa[0] == pb[0]) & (pa[1] < pb[1]))
    return tuple(jnp.where(t, u, v) for u, v in zip(pa, pb))


def _bcast_rows(t, ones_mat, nrows):
    hi = t.astype(jnp.bfloat16).astype(jnp.float32)
    r1 = t - hi
    mid = r1.astype(jnp.bfloat16).astype(jnp.float32)
    lo = r1 - mid
    big = jnp.concatenate([hi, mid, lo], axis=0)
    bb = jax.lax.dot_general(
        big, ones_mat, (((1,), (0,)), ((), ())),
        preferred_element_type=jnp.float32,
    )
    return bb[0:nrows] + bb[nrows:2 * nrows] + bb[2 * nrows:3 * nrows]


def _nms_body(x1_ref, y1_ref, x2_ref, y2_ref, s_ref, out_ref):
    x1 = x1_ref[...]
    y1 = y1_ref[...]
    x2 = x2_ref[...]
    y2 = y2_ref[...]
    scores = s_ref[...]
    s0 = jnp.where(scores > _SCORE_THRESH, scores, _NEG)
    area3 = (x2 - x1) * (y2 - y1) * (1.0 / 3.0)
    rowid = jax.lax.broadcasted_iota(jnp.int32, (_ROWS, _COLS), 0)
    lane = jax.lax.broadcasted_iota(jnp.int32, (1, _COLS), 1)
    ones_mat = jnp.ones((_COLS, _COLS), jnp.float32)
    nchunk = _ROWS // 8

    sm = s0
    slots = []
    for _ in range(_POOL):
        cmax = jnp.max(sm, axis=0, keepdims=True)
        rsel = jnp.min(
            jnp.where(sm == cmax, rowid, _ROWS), axis=0, keepdims=True
        )
        hot = rowid == rsel
        px1 = jnp.sum(jnp.where(hot, x1, 0.0), axis=0, keepdims=True)
        py1 = jnp.sum(jnp.where(hot, y1, 0.0), axis=0, keepdims=True)
        px2 = jnp.sum(jnp.where(hot, x2, 0.0), axis=0, keepdims=True)
        py2 = jnp.sum(jnp.where(hot, y2, 0.0), axis=0, keepdims=True)
        slots.append((cmax, rsel.astype(jnp.float32), px1, py1, px2, py2))
        sm = jnp.where(hot, _NEG, sm)
    t_next = jnp.max(sm, axis=0, keepdims=True)
    t_next = jnp.max(t_next, axis=1, keepdims=True)

    pool_s = jnp.concatenate([sl[0] for sl in slots], axis=0)
    pool_r = jnp.concatenate([sl[1] for sl in slots], axis=0)
    pool_x1 = jnp.concatenate([sl[2] for sl in slots], axis=0)
    pool_y1 = jnp.concatenate([sl[3] for sl in slots], axis=0)
    pool_x2 = jnp.concatenate([sl[4] for sl in slots], axis=0)
    pool_y2 = jnp.concatenate([sl[5] for sl in slots], axis=0)
    pool_a3 = (pool_x2 - pool_x1) * (pool_y2 - pool_y1) * (1.0 / 3.0)

    def fast_body(i, st):
        ps, pois = st
        pair = (ps, pool_r, pool_x1, pool_y1, pool_x2, pool_y2)
        w4 = _take(tuple(u[0:4] for u in pair), tuple(u[4:8] for u in pair))
        w2 = _take(tuple(u[0:2] for u in w4), tuple(u[2:4] for u in w4))
        w1 = _take(tuple(u[0:1] for u in w2), tuple(u[1:2] for u in w2))
        colv, colr, colx1, coly1, colx2, coly2 = w1

        c = jnp.argmax(colv, axis=1).reshape(1, 1)
        lane_hot = lane == c
        stack = jnp.concatenate(
            [colx1, coly1, colx2, coly2, colv, colr], axis=0
        )
        t = jnp.where(lane_hot, stack, 0.0)
        b = _bcast_rows(t, ones_mat, 6)
        bx1 = b[0:1, :]
        by1 = b[1:2, :]
        bx2 = b[2:3, :]
        by2 = b[3:4, :]
        m = b[4:5, :]
        wrow = b[5:6, :]
        barea3 = (bx2 - bx1) * (by2 - by1) * (1.0 / 3.0)
        valid = m > _NEG / 2.0

        iw = jnp.maximum(
            jnp.minimum(bx2, pool_x2) - jnp.maximum(bx1, pool_x1), 0.0
        )
        ih = jnp.maximum(
            jnp.minimum(by2, pool_y2) - jnp.maximum(by1, pool_y1), 0.0
        )
        inter = iw * ih
        suppress = inter > pool_a3 + (barea3 + 1e-9 / 3.0)
        retire = lane_hot & (ps == m) & (pool_r == wrow)
        ps = jnp.where(suppress | retire, _NEG, ps)

        bad = (m[0:1, 0:1] <= t_next) & (t_next > _NEG / 2.0)
        pois = jnp.maximum(pois, bad.astype(jnp.float32))

        row = (
            jnp.where(lane == 0, bx1, 0.0)
            + jnp.where(lane == 1, by1, 0.0)
            + jnp.where(lane == 2, bx2, 0.0)
            + jnp.where(lane == 3, by2, 0.0)
            + jnp.where(lane == 4, m, 0.0)
        )
        out_ref[pl.ds(i, 1), :] = jnp.where(valid, row, 0.0)
        return ps, pois

    pois0 = jnp.zeros((1, 1), jnp.float32)
    _, pois = jax.lax.fori_loop(
        0, _K, fast_body, (pool_s, pois0), unroll=False
    )

    @pl.when(pois.astype(jnp.int32)[0, 0] != 0)
    def _slow():
        def body(i, s):
            ps = [
                (s[k * 8:(k + 1) * 8], rowid[k * 8:(k + 1) * 8])
                for k in range(nchunk)
            ]
            while len(ps) > 1:
                nxt = []
                for a in range(0, len(ps) - 1, 2):
                    nxt.append(_take(ps[a], ps[a + 1]))
                if len(ps) % 2:
                    nxt.append(ps[-1])
                ps = nxt
            w8 = ps[0]
            w4 = _take(tuple(u[0:4] for u in w8), tuple(u[4:8] for u in w8))
            w2 = _take(tuple(u[0:2] for u in w4), tuple(u[2:4] for u in w4))
            w1 = _take(tuple(u[0:1] for u in w2), tuple(u[1:2] for u in w2))
            colv, colrow = w1

            hot2d = rowid == colrow
            colx1 = jnp.sum(jnp.where(hot2d, x1, 0.0), axis=0, keepdims=True)
            coly1 = jnp.sum(jnp.where(hot2d, y1, 0.0), axis=0, keepdims=True)
            colx2 = jnp.sum(jnp.where(hot2d, x2, 0.0), axis=0, keepdims=True)
            coly2 = jnp.sum(jnp.where(hot2d, y2, 0.0), axis=0, keepdims=True)

            c = jnp.argmax(colv, axis=1).reshape(1, 1)
            lane_hot = lane == c
            stack = jnp.concatenate(
                [colx1, coly1, colx2, coly2, colv], axis=0
            )
            t = jnp.where(lane_hot, stack, 0.0)
            b = _bcast_rows(t, ones_mat, 5)
            bx1 = b[0:1, :]
            by1 = b[1:2, :]
            bx2 = b[2:3, :]
            by2 = b[3:4, :]
            m = b[4:5, :]
            barea3 = (bx2 - bx1) * (by2 - by1) * (1.0 / 3.0)
            valid = m > _NEG / 2.0

            iw = jnp.maximum(jnp.minimum(bx2, x2) - jnp.maximum(bx1, x1), 0.0)
            ih = jnp.maximum(jnp.minimum(by2, y2) - jnp.maximum(by1, y1), 0.0)
            inter = iw * ih
            suppress = inter > area3 + (barea3 + 1e-9 / 3.0)
            s = jnp.where(suppress, _NEG, s)

            row = (
                jnp.where(lane == 0, bx1, 0.0)
                + jnp.where(lane == 1, by1, 0.0)
                + jnp.where(lane == 2, bx2, 0.0)
                + jnp.where(lane == 3, by2, 0.0)
                + jnp.where(lane == 4, m, 0.0)
            )
            out_ref[pl.ds(i, 1), :] = jnp.where(valid, row, 0.0)
            return s

        jax.lax.fori_loop(0, _K, body, s0, unroll=False)


def _to_layout(v, fill):
    v = jnp.pad(v, (0, _N_PAD - _N), constant_values=fill)
    return v.reshape(_COLS, _ROWS).T[:, ::-1]


def kernel(boxes, scores):
    x1 = _to_layout(boxes[:, 0], 0.0)
    y1 = _to_layout(boxes[:, 1], 0.0)
    x2 = _to_layout(boxes[:, 2], 0.0)
    y2 = _to_layout(boxes[:, 3], 0.0)
    s = _to_layout(scores, -1.0)

    out = pl.pallas_call(
        _nms_body,
        out_shape=jax.ShapeDtypeStruct((_K, _COLS), jnp.float32),
        in_specs=[pl.BlockSpec(memory_space=pltpu.VMEM)] * 5,
        out_specs=pl.BlockSpec(memory_space=pltpu.VMEM),
    )(x1, y1, x2, y2, s)
    return out[:, :5]
